# baseline (speedup 1.0000x reference)
import jax
import jax.numpy as jnp
from jax import lax
from jax.experimental import pallas as pl
from jax.experimental.pallas import tpu as pltpu
from jax.experimental.pallas import tpu_sc as plsc

BATCH = 16384
HIST = 200
EMB = 16
N = BATCH * HIST

NUM_CORES = 2
NUM_SUBCORES = 16
NW = NUM_CORES * NUM_SUBCORES
PER_W = N // NW
K = 4
SUB = 512
CHUNK = K * SUB
NC = PER_W // CHUNK
PAIRS = NC // 2
ROWS2D = N // SUB
ROWS_PER_W = PER_W // SUB


def _body(x_hbm, table_hbm, out_hbm, idx_v, rows_v,
          ix_sem0, ix_sem1, g_sem0, g_sem1, st_sem0, st_sem1):
    wid = lax.axis_index("s") * NUM_CORES + lax.axis_index("c")
    base = wid * PER_W
    base_row = wid * ROWS_PER_W
    ix_sems = (ix_sem0, ix_sem1)
    g_sems = (g_sem0, g_sem1)
    st_sems = (st_sem0, st_sem1)

    def idx_copy(b, c):
        return pltpu.make_async_copy(
            x_hbm.at[pl.ds(base_row + c * K, K)], idx_v.at[b], ix_sems[b])

    def sub_gather(b, j):
        return pltpu.make_async_copy(
            table_hbm.at[idx_v.at[b, j]], rows_v.at[b, j], g_sems[b])

    def sub_store(b, j, c):
        return pltpu.make_async_copy(
            rows_v.at[b, j],
            out_hbm.at[pl.ds(base + c * CHUNK + j * SUB, SUB)], st_sems[b])

    def gather_start(b):
        for j in range(K):
            sub_gather(b, j).start()

    def gather_wait(b):
        for j in range(K):
            sub_gather(b, j).wait()

    def store_start(b, c):
        for j in range(K):
            sub_store(b, j, c).start()

    def store_wait(b, c):
        for j in range(K):
            sub_store(b, j, c).wait()

    def slot(b, c, first, last, head=False):
        if not first:
            store_wait(b, c - 2)
        idx_copy(b, c).wait()
        gather_start(b)
        if not head:
            gather_wait(1 - b)
            store_start(1 - b, c - 1)
        if not last:
            idx_copy(1 - b, c + 1).start()

    idx_copy(0, 0).start()
    slot(0, 0, first=True, last=False, head=True)
    slot(1, 1, first=True, last=False)

    def pair(t, carry):
        c0 = t * 2
        slot(0, c0, first=False, last=False)
        slot(1, c0 + 1, first=False, last=False)
        return carry

    lax.fori_loop(1, PAIRS - 1, pair, 0, unroll=False)

    c0 = NC - 2
    slot(0, c0, first=False, last=False)
    slot(1, c0 + 1, first=False, last=True)
    gather_wait(1)
    store_start(1, NC - 1)
    store_wait(0, NC - 2)
    store_wait(1, NC - 1)


@jax.jit
def _lookup(x2d, table):
    mesh = plsc.VectorSubcoreMesh(core_axis_name="c", subcore_axis_name="s")
    return pl.kernel(
        _body,
        out_type=jax.ShapeDtypeStruct((N, EMB), jnp.float32),
        mesh=mesh,
        scratch_types=[
            pltpu.VMEM((2, K, SUB), jnp.int32),
            pltpu.VMEM((2, K, SUB, EMB), jnp.float32),
            pltpu.SemaphoreType.DMA,
            pltpu.SemaphoreType.DMA,
            pltpu.SemaphoreType.DMA,
            pltpu.SemaphoreType.DMA,
            pltpu.SemaphoreType.DMA,
            pltpu.SemaphoreType.DMA,
        ],
        compiler_params=pltpu.CompilerParams(use_tc_tiling_on_sc=False),
    )(x2d, table)


def kernel(x, table):
    x2d = x.reshape(ROWS2D, SUB).astype(jnp.int32)
    out = _lookup(x2d, table)
    return out.reshape(BATCH, HIST, EMB)
